# baseline (speedup 1.0000x reference)
import functools

import jax
import jax.numpy as jnp
from jax.experimental import pallas as pl

MODEL_DIM = 2048
NUM_EXPERTS = 16
K = 2
N_TOKENS = 16384
BLOCK_TOKENS = 512


def _gate_kernel(x_ref, w_ref, b_ref, idx_ref, score_ref):
    x = x_ref[...]
    w = w_ref[...]
    logits = jnp.dot(x, w, preferred_element_type=jnp.float32)
    logits = logits + b_ref[...]

    iota = jax.lax.broadcasted_iota(jnp.int32, logits.shape, 1)
    big = jnp.int32(NUM_EXPERTS)

    m1 = jnp.max(logits, axis=1, keepdims=True)
    i1 = jnp.min(jnp.where(logits == m1, iota, big), axis=1, keepdims=True)
    masked = jnp.where(iota == i1, -jnp.inf, logits)
    m2 = jnp.max(masked, axis=1, keepdims=True)
    i2 = jnp.min(jnp.where(masked == m2, iota, big), axis=1, keepdims=True)

    e = jnp.exp(m2 - m1)
    denom = 1.0 + e
    s1 = 1.0 / denom
    s2 = e / denom

    idx_ref[...] = jnp.concatenate([i1, i2], axis=1)
    score_ref[...] = jnp.concatenate([s1, s2], axis=1)


@jax.jit
def kernel(x, W, b):
    n = x.shape[0]
    grid = (n // BLOCK_TOKENS,)
    b2 = b.reshape(1, NUM_EXPERTS)
    wt = W.T
    idx, scores = pl.pallas_call(
        _gate_kernel,
        grid=grid,
        in_specs=[
            pl.BlockSpec((BLOCK_TOKENS, MODEL_DIM), lambda i: (i, 0)),
            pl.BlockSpec((MODEL_DIM, NUM_EXPERTS), lambda i: (0, 0)),
            pl.BlockSpec((1, NUM_EXPERTS), lambda i: (0, 0)),
        ],
        out_specs=[
            pl.BlockSpec((BLOCK_TOKENS, K), lambda i: (i, 0)),
            pl.BlockSpec((BLOCK_TOKENS, K), lambda i: (i, 0)),
        ],
        out_shape=[
            jax.ShapeDtypeStruct((n, K), jnp.int32),
            jax.ShapeDtypeStruct((n, K), jnp.float32),
        ],
    )(x, wt, b2)
    return (idx, scores)


# speedup vs baseline: 1.2248x; 1.2248x over previous
import functools

import jax
import jax.numpy as jnp
from jax.experimental import pallas as pl

MODEL_DIM = 2048
NUM_EXPERTS = 16
K = 2
N_TOKENS = 16384
BLOCK_TOKENS = 1024


def _gate_kernel(x_ref, w_ref, b_ref, idx_ref, score_ref):
    x = x_ref[...]
    w = w_ref[...]
    logits = jax.lax.dot_general(
        x, w, (((1,), (1,)), ((), ())), preferred_element_type=jnp.float32
    )
    logits = logits + b_ref[...]

    iota = jax.lax.broadcasted_iota(jnp.int32, logits.shape, 1)
    big = jnp.int32(NUM_EXPERTS)

    m1 = jnp.max(logits, axis=1, keepdims=True)
    i1 = jnp.min(jnp.where(logits == m1, iota, big), axis=1, keepdims=True)
    masked = jnp.where(iota == i1, -jnp.inf, logits)
    m2 = jnp.max(masked, axis=1, keepdims=True)
    i2 = jnp.min(jnp.where(masked == m2, iota, big), axis=1, keepdims=True)

    e = jnp.exp(m2 - m1)
    denom = 1.0 + e
    s1 = 1.0 / denom
    s2 = e / denom

    idx_ref[...] = jnp.concatenate([i1, i2], axis=1)
    score_ref[...] = jnp.concatenate([s1, s2], axis=1)


@jax.jit
def kernel(x, W, b):
    n = x.shape[0]
    grid = (n // BLOCK_TOKENS,)
    b2 = b.reshape(1, NUM_EXPERTS)
    idx, scores = pl.pallas_call(
        _gate_kernel,
        grid=grid,
        in_specs=[
            pl.BlockSpec((BLOCK_TOKENS, MODEL_DIM), lambda i: (i, 0)),
            pl.BlockSpec((NUM_EXPERTS, MODEL_DIM), lambda i: (0, 0)),
            pl.BlockSpec((1, NUM_EXPERTS), lambda i: (0, 0)),
        ],
        out_specs=[
            pl.BlockSpec((BLOCK_TOKENS, K), lambda i: (i, 0)),
            pl.BlockSpec((BLOCK_TOKENS, K), lambda i: (i, 0)),
        ],
        out_shape=[
            jax.ShapeDtypeStruct((n, K), jnp.int32),
            jax.ShapeDtypeStruct((n, K), jnp.float32),
        ],
    )(x, W, b2)
    return (idx, scores)


# speedup vs baseline: 1.2608x; 1.0294x over previous
import functools

import jax
import jax.numpy as jnp
from jax.experimental import pallas as pl

MODEL_DIM = 2048
NUM_EXPERTS = 16
K = 2
N_TOKENS = 16384
BLOCK_TOKENS = 2048


def _gate_kernel(x_ref, w_ref, b_ref, idx_ref, score_ref):
    x = x_ref[...]
    w = w_ref[...]
    logits = jax.lax.dot_general(
        x, w, (((1,), (1,)), ((), ())), preferred_element_type=jnp.float32
    )
    logits = logits + b_ref[...]

    iota = jax.lax.broadcasted_iota(jnp.int32, logits.shape, 1)
    big = jnp.int32(NUM_EXPERTS)

    m1 = jnp.max(logits, axis=1, keepdims=True)
    i1 = jnp.min(jnp.where(logits == m1, iota, big), axis=1, keepdims=True)
    masked = jnp.where(iota == i1, -jnp.inf, logits)
    m2 = jnp.max(masked, axis=1, keepdims=True)
    i2 = jnp.min(jnp.where(masked == m2, iota, big), axis=1, keepdims=True)

    e = jnp.exp(m2 - m1)
    denom = 1.0 + e
    s1 = 1.0 / denom
    s2 = e / denom

    idx_ref[...] = jnp.concatenate([i1, i2], axis=1)
    score_ref[...] = jnp.concatenate([s1, s2], axis=1)


@jax.jit
def kernel(x, W, b):
    n = x.shape[0]
    grid = (n // BLOCK_TOKENS,)
    b2 = b.reshape(1, NUM_EXPERTS)
    idx, scores = pl.pallas_call(
        _gate_kernel,
        grid=grid,
        in_specs=[
            pl.BlockSpec((BLOCK_TOKENS, MODEL_DIM), lambda i: (i, 0)),
            pl.BlockSpec((NUM_EXPERTS, MODEL_DIM), lambda i: (0, 0)),
            pl.BlockSpec((1, NUM_EXPERTS), lambda i: (0, 0)),
        ],
        out_specs=[
            pl.BlockSpec((BLOCK_TOKENS, K), lambda i: (i, 0)),
            pl.BlockSpec((BLOCK_TOKENS, K), lambda i: (i, 0)),
        ],
        out_shape=[
            jax.ShapeDtypeStruct((n, K), jnp.int32),
            jax.ShapeDtypeStruct((n, K), jnp.float32),
        ],
    )(x, W, b2)
    return (idx, scores)


# speedup vs baseline: 1.6846x; 1.3361x over previous
import functools

import jax
import jax.numpy as jnp
from jax.experimental import pallas as pl

MODEL_DIM = 2048
NUM_EXPERTS = 16
K = 2
N_TOKENS = 16384
BLOCK_TOKENS = 2048


def _gate_kernel(x_ref, w_ref, b_ref, idx_ref, score_ref):
    x = x_ref[...]
    w = w_ref[...]
    logits = jax.lax.dot_general(
        w, x, (((1,), (1,)), ((), ())), preferred_element_type=jnp.float32
    )
    logits = logits + b_ref[...]

    iota = jax.lax.broadcasted_iota(jnp.int32, logits.shape, 0)
    big = jnp.int32(NUM_EXPERTS)

    m1 = jnp.max(logits, axis=0, keepdims=True)
    i1 = jnp.min(jnp.where(logits == m1, iota, big), axis=0, keepdims=True)
    masked = jnp.where(iota == i1, -jnp.inf, logits)
    m2 = jnp.max(masked, axis=0, keepdims=True)
    i2 = jnp.min(jnp.where(masked == m2, iota, big), axis=0, keepdims=True)

    e = jnp.exp(m2 - m1)
    denom = 1.0 + e
    s1 = 1.0 / denom
    s2 = e / denom

    idx_ref[...] = jnp.concatenate([i1, i2], axis=0)
    score_ref[...] = jnp.concatenate([s1, s2], axis=0)


@jax.jit
def kernel(x, W, b):
    n = x.shape[0]
    grid = (n // BLOCK_TOKENS,)
    b2 = b.reshape(NUM_EXPERTS, 1)
    idx_t, scores_t = pl.pallas_call(
        _gate_kernel,
        grid=grid,
        in_specs=[
            pl.BlockSpec((BLOCK_TOKENS, MODEL_DIM), lambda i: (i, 0)),
            pl.BlockSpec((NUM_EXPERTS, MODEL_DIM), lambda i: (0, 0)),
            pl.BlockSpec((NUM_EXPERTS, 1), lambda i: (0, 0)),
        ],
        out_specs=[
            pl.BlockSpec((K, BLOCK_TOKENS), lambda i: (0, i)),
            pl.BlockSpec((K, BLOCK_TOKENS), lambda i: (0, i)),
        ],
        out_shape=[
            jax.ShapeDtypeStruct((K, n), jnp.int32),
            jax.ShapeDtypeStruct((K, n), jnp.float32),
        ],
    )(x, W, b2)
    return (idx_t.T, scores_t.T)


# speedup vs baseline: 1.7810x; 1.0572x over previous
import functools

import jax
import jax.numpy as jnp
from jax.experimental import pallas as pl

MODEL_DIM = 2048
NUM_EXPERTS = 16
K = 2
N_TOKENS = 16384
BLOCK_TOKENS = 1024


def _gate_kernel(x_ref, w_ref, b_ref, idx_ref, score_ref):
    x = x_ref[...]
    w = w_ref[...]
    logits = jax.lax.dot_general(
        w, x, (((1,), (1,)), ((), ())), preferred_element_type=jnp.float32
    )
    logits = logits + b_ref[...]

    iota = jax.lax.broadcasted_iota(jnp.int32, logits.shape, 0)
    big = jnp.int32(NUM_EXPERTS)

    m1 = jnp.max(logits, axis=0, keepdims=True)
    i1 = jnp.min(jnp.where(logits == m1, iota, big), axis=0, keepdims=True)
    masked = jnp.where(iota == i1, -jnp.inf, logits)
    m2 = jnp.max(masked, axis=0, keepdims=True)
    i2 = jnp.min(jnp.where(masked == m2, iota, big), axis=0, keepdims=True)

    e = jnp.exp(m2 - m1)
    denom = 1.0 + e
    s1 = 1.0 / denom
    s2 = e / denom

    idx_ref[...] = jnp.concatenate([i1, i2], axis=0)
    score_ref[...] = jnp.concatenate([s1, s2], axis=0)


@jax.jit
def kernel(x, W, b):
    n = x.shape[0]
    grid = (n // BLOCK_TOKENS,)
    b2 = b.reshape(NUM_EXPERTS, 1)
    idx_t, scores_t = pl.pallas_call(
        _gate_kernel,
        grid=grid,
        in_specs=[
            pl.BlockSpec((BLOCK_TOKENS, MODEL_DIM), lambda i: (i, 0)),
            pl.BlockSpec((NUM_EXPERTS, MODEL_DIM), lambda i: (0, 0)),
            pl.BlockSpec((NUM_EXPERTS, 1), lambda i: (0, 0)),
        ],
        out_specs=[
            pl.BlockSpec((K, BLOCK_TOKENS), lambda i: (0, i)),
            pl.BlockSpec((K, BLOCK_TOKENS), lambda i: (0, i)),
        ],
        out_shape=[
            jax.ShapeDtypeStruct((K, n), jnp.int32),
            jax.ShapeDtypeStruct((K, n), jnp.float32),
        ],
    )(x, W, b2)
    return (idx_t.T, scores_t.T)
